# baseline (speedup 1.0000x reference)
import jax
import jax.numpy as jnp
from jax.experimental import pallas as pl
from jax.experimental.pallas import tpu as pltpu

_B = 4096
_S = 200
_D = 64
_BLK = 256


def _sum_body(x_ref, o_ref):
    o_ref[...] = jnp.sum(x_ref[...], axis=1)


def kernel(inputs):
    grid = (_B // _BLK,)
    return pl.pallas_call(
        _sum_body,
        grid=grid,
        in_specs=[pl.BlockSpec((_BLK, _S, _D), lambda i: (i, 0, 0))],
        out_specs=pl.BlockSpec((_BLK, _D), lambda i: (i, 0)),
        out_shape=jax.ShapeDtypeStruct((_B, _D), jnp.float32),
    )(inputs)


# speedup vs baseline: 1.6389x; 1.6389x over previous
import jax
import jax.numpy as jnp
from jax.experimental import pallas as pl

_B = 4096
_S = 200
_D = 64
_BLK = 512
_NSLICE = (_S * _D) // 128


def _sum_body(x_ref, o_ref):
    acc = x_ref[:, 0:128]
    for j in range(1, _NSLICE):
        acc = acc + x_ref[:, j * 128:(j + 1) * 128]
    o_ref[...] = acc[:, :_D] + acc[:, _D:]


def kernel(inputs):
    x = inputs.reshape(_B, _S * _D)
    return pl.pallas_call(
        _sum_body,
        grid=(_B // _BLK,),
        in_specs=[pl.BlockSpec((_BLK, _S * _D), lambda i: (i, 0))],
        out_specs=pl.BlockSpec((_BLK, _D), lambda i: (i, 0)),
        out_shape=jax.ShapeDtypeStruct((_B, _D), jnp.float32),
    )(x)


# speedup vs baseline: 6.5717x; 4.0099x over previous
import jax
import jax.numpy as jnp
from jax.experimental import pallas as pl

_B = 4096
_S = 200
_D = 64
_BLKL = 256


def _sum_body(x_ref, o_ref):
    o_ref[...] = jnp.sum(x_ref[...], axis=0)


def kernel(inputs):
    x = jnp.transpose(inputs, (1, 2, 0))
    out_t = pl.pallas_call(
        _sum_body,
        grid=(_B // _BLKL,),
        in_specs=[pl.BlockSpec((_S, _D, _BLKL), lambda i: (0, 0, i))],
        out_specs=pl.BlockSpec((_D, _BLKL), lambda i: (0, i)),
        out_shape=jax.ShapeDtypeStruct((_D, _B), jnp.float32),
    )(x)
    return jnp.transpose(out_t, (1, 0))
